# baseline (speedup 1.0000x reference)
import functools

import jax
import jax.numpy as jnp
from jax import lax
from jax.experimental import pallas as pl
from jax.experimental.pallas import tpu as pltpu
from jax.experimental.pallas import tpu_sc as plsc

N_USER_C = 50000
N_ITEM_C = 50000
N_C = N_USER_C + N_ITEM_C
E_C = 1600000
D_C = 32
HD = 16
B_C = 4096
LAYERS_C = 3

NC = 2
NS = 16

GSZ = 128
GROUPS_PER_TILE = 800
E_PAD = NS * GROUPS_PER_TILE * GSZ
G_TOTAL = E_PAD // GSZ
CHUNK_G = 20
CHUNKS = GROUPS_PER_TILE // CHUNK_G

STRIPE = 6256
N_ZERO = (STRIPE + GSZ - 1) // GSZ

_mesh = plsc.VectorSubcoreMesh(core_axis_name="c", subcore_axis_name="s")
_cparams = pltpu.CompilerParams(use_tc_tiling_on_sc=False)


def _scale(vbuf, jj, rb):
    for v in range(GSZ // 16):
        wv = vbuf[jj, pl.ds(v * 16, 16)]
        for ee in range(16):
            w = jnp.broadcast_to(wv[ee:ee + 1], (16,))
            e = v * 16 + ee
            rb[e, pl.ds(0, HD)] = rb[e, pl.ds(0, HD)] * w


def _spmm_half(sid, ego, out, epk, vpk, acc, eb0, eb1, vb0, vb1,
               rbs, zbuf, sem_e, gsem, ssem):
    @pl.loop(0, GSZ)
    def _(e):
        zbuf[e, pl.ds(0, HD)] = jnp.zeros((HD,), jnp.float32)

    base = jnp.minimum(sid * STRIPE, N_C - STRIPE)

    @pl.loop(0, N_ZERO)
    def _(g):
        off = jnp.minimum(g * GSZ, STRIPE - GSZ)
        pltpu.sync_copy(zbuf, acc.at[pl.ds(base + off, GSZ)])

    plsc.subcore_barrier()

    gb0 = sid * GROUPS_PER_TILE
    pltpu.async_copy(epk.at[pl.ds(gb0, CHUNK_G)], eb0, sem_e)
    pltpu.async_copy(vpk.at[pl.ds(gb0, CHUNK_G)], vb0, sem_e)

    def do_chunk(c, ebuf, ebnext, vbuf, vbnext):
        def scat_wait(b):
            pltpu.make_async_copy(
                rbs[b], acc.at[ebuf.at[0, 0]], ssem[b]).wait()

        pltpu.make_async_copy(epk.at[pl.ds(0, CHUNK_G)], ebuf, sem_e).wait()
        pltpu.make_async_copy(vpk.at[pl.ds(0, CHUNK_G)], vbuf, sem_e).wait()

        @pl.when(c + 1 < CHUNKS)
        def _():
            nb = gb0 + (c + 1) * CHUNK_G
            pltpu.async_copy(epk.at[pl.ds(nb, CHUNK_G)], ebnext, sem_e)
            pltpu.async_copy(vpk.at[pl.ds(nb, CHUNK_G)], vbnext, sem_e)

        pltpu.async_copy(ego.at[ebuf.at[0, 1]], rbs[0], gsem[0])
        pltpu.async_copy(ego.at[ebuf.at[1, 1]], rbs[1], gsem[1])

        @pl.loop(0, CHUNK_G // 4)
        def _(i):
            for k in range(4):
                q = 4 * i + k
                nxt = (k + 2) % 4
                prv = (k + 3) % 4
                if k == 0:
                    @pl.when(i > 0)
                    def _():
                        scat_wait(prv)
                else:
                    scat_wait(prv)

                @pl.when(q + 2 < CHUNK_G)
                def _():
                    pltpu.async_copy(
                        ego.at[ebuf.at[q + 2, 1]], rbs[nxt], gsem[nxt])

                pltpu.make_async_copy(
                    ego.at[pl.ds(0, GSZ)], rbs[k], gsem[k]).wait()
                _scale(vbuf, q, rbs[k])
                pltpu.async_copy(
                    rbs[k], acc.at[ebuf.at[q, 0]], ssem[k], add=True)

        scat_wait(3)

    @pl.loop(0, CHUNKS // 2)
    def _(c2):
        do_chunk(2 * c2, eb0, eb1, vb0, vb1)
        do_chunk(2 * c2 + 1, eb1, eb0, vb1, vb0)

    plsc.subcore_barrier()
    pltpu.sync_copy(acc.at[pl.ds(base, STRIPE)], out.at[pl.ds(base, STRIPE)])


@functools.partial(
    pl.kernel,
    out_type=[jax.ShapeDtypeStruct((N_C, HD), jnp.float32)] * 2,
    mesh=_mesh,
    scratch_types=[
        pltpu.VMEM_SHARED((N_C, HD), jnp.float32),
        pltpu.VMEM((CHUNK_G, 2, GSZ), jnp.int32),
        pltpu.VMEM((CHUNK_G, 2, GSZ), jnp.int32),
        pltpu.VMEM((CHUNK_G, GSZ), jnp.float32),
        pltpu.VMEM((CHUNK_G, GSZ), jnp.float32),
        pltpu.VMEM((GSZ, HD), jnp.float32),
        pltpu.VMEM((GSZ, HD), jnp.float32),
        pltpu.VMEM((GSZ, HD), jnp.float32),
        pltpu.VMEM((GSZ, HD), jnp.float32),
        pltpu.VMEM((GSZ, HD), jnp.float32),
        pltpu.SemaphoreType.DMA,
        pltpu.SemaphoreType.DMA,
        pltpu.SemaphoreType.DMA,
        pltpu.SemaphoreType.DMA,
        pltpu.SemaphoreType.DMA,
        pltpu.SemaphoreType.DMA,
        pltpu.SemaphoreType.DMA,
        pltpu.SemaphoreType.DMA,
        pltpu.SemaphoreType.DMA,
    ],
    compiler_params=_cparams,
)
def _spmm(egoA, egoB, epk, vpk, outA, outB,
          acc, eb0, eb1, vb0, vb1, rb0, rb1, rb2, rb3, zbuf, sem_e,
          sg0, sg1, sg2, sg3, ss0, ss1, ss2, ss3):
    cid = lax.axis_index("c")
    sid = lax.axis_index("s")
    rbs = (rb0, rb1, rb2, rb3)
    gsem = (sg0, sg1, sg2, sg3)
    ssem = (ss0, ss1, ss2, ss3)

    @pl.when(cid == 0)
    def _():
        _spmm_half(sid, egoA, outA, epk, vpk, acc, eb0, eb1, vb0, vb1,
                   rbs, zbuf, sem_e, gsem, ssem)

    @pl.when(cid == 1)
    def _():
        _spmm_half(sid, egoB, outB, epk, vpk, acc, eb0, eb1, vb0, vb1,
                   rbs, zbuf, sem_e, gsem, ssem)


IDX_TOTAL = 3 * B_C
IDX_G = IDX_TOTAL // GSZ
IDX_G_PER_TILE = 8
IDX_TILES = IDX_G // IDX_G_PER_TILE


@functools.partial(
    pl.kernel,
    out_type=[jax.ShapeDtypeStruct((IDX_TOTAL, HD), jnp.float32)] * (2 * (LAYERS_C + 1)),
    mesh=_mesh,
    scratch_types=[
        pltpu.VMEM((IDX_G_PER_TILE, GSZ), jnp.int32),
        pltpu.VMEM((GSZ, HD), jnp.float32),
    ],
    compiler_params=_cparams,
)
def _gather8(t0, t1, t2, t3, t4, t5, t6, t7, idx_hbm,
             o0, o1, o2, o3, o4, o5, o6, o7, idxv, buf):
    cid = lax.axis_index("c")
    sid = lax.axis_index("s")
    w = cid * NS + sid

    @pl.when(w < IDX_TILES)
    def _():
        pltpu.sync_copy(
            idx_hbm.at[pl.ds(w * IDX_G_PER_TILE, IDX_G_PER_TILE)], idxv)
        for tab, out in ((t0, o0), (t1, o1), (t2, o2), (t3, o3),
                         (t4, o4), (t5, o5), (t6, o6), (t7, o7)):
            @pl.loop(0, IDX_G_PER_TILE)
            def _(j):
                pltpu.sync_copy(tab.at[idxv.at[j]], buf)
                pltpu.sync_copy(
                    buf, out.at[pl.ds((w * IDX_G_PER_TILE + j) * GSZ, GSZ)])


def kernel(users, pos_items, neg_items, edge_index, adj_vals, user_emb, item_emb):
    ego0 = jnp.concatenate([user_emb, item_emb], axis=0)
    row = edge_index[0].astype(jnp.int32)
    col = edge_index[1].astype(jnp.int32)
    pad = E_PAD - E_C
    rowp = jnp.pad(row, (0, pad)).reshape(G_TOTAL, GSZ)
    colp = jnp.pad(col, (0, pad)).reshape(G_TOTAL, GSZ)
    vpk = jnp.pad(adj_vals, (0, pad)).reshape(G_TOTAL, GSZ)
    epk = jnp.stack([rowp, colp], axis=1)

    halves = [(ego0[:, :HD], ego0[:, HD:])]
    for _ in range(LAYERS_C):
        halves.append(tuple(_spmm(halves[-1][0], halves[-1][1], epk, vpk)))

    idx_all = jnp.concatenate([
        users.astype(jnp.int32),
        pos_items.astype(jnp.int32) + N_USER_C,
        neg_items.astype(jnp.int32) + N_USER_C,
    ]).reshape(IDX_G, GSZ)

    tabs = [h for pair in halves for h in pair]
    g = _gather8(*tabs, idx_all)
    cat = jnp.concatenate(
        [jnp.concatenate([g[2 * k], g[2 * k + 1]], axis=1)
         for k in range(LAYERS_C + 1)], axis=1)
    return (cat[:B_C], cat[B_C:2 * B_C], cat[2 * B_C:])


# speedup vs baseline: 1.0902x; 1.0902x over previous
import functools

import jax
import jax.numpy as jnp
from jax import lax
from jax.experimental import pallas as pl
from jax.experimental.pallas import tpu as pltpu
from jax.experimental.pallas import tpu_sc as plsc

N_USER_C = 50000
N_ITEM_C = 50000
N_C = N_USER_C + N_ITEM_C
E_C = 1600000
D_C = 32
HD = 16
B_C = 4096
LAYERS_C = 3

NC = 2
NS = 16

GSZ = 128
GROUPS_PER_TILE = 800
E_PAD = NS * GROUPS_PER_TILE * GSZ
G_TOTAL = E_PAD // GSZ
CHUNK_G = 20
CHUNKS = GROUPS_PER_TILE // CHUNK_G

STRIPE = 6256
N_ZERO = (STRIPE + GSZ - 1) // GSZ

_mesh = plsc.VectorSubcoreMesh(core_axis_name="c", subcore_axis_name="s")
_cparams = pltpu.CompilerParams(use_tc_tiling_on_sc=False)


def _scale(vbuf, jj, rb):
    @pl.loop(0, GSZ // 16)
    def _(v):
        wv = vbuf[jj, pl.ds(v * 16, 16)]
        for ee in range(16):
            w = jnp.broadcast_to(wv[ee:ee + 1], (16,))
            e = v * 16 + ee
            rb[e, pl.ds(0, HD)] = rb[e, pl.ds(0, HD)] * w


def _layer(sid, ego, out, epk, vpk, acc, eb0, eb1, vb0, vb1,
           rbs, zbuf, sem_e, gsem, ssem):
    @pl.loop(0, GSZ)
    def _(e):
        zbuf[e, pl.ds(0, HD)] = jnp.zeros((HD,), jnp.float32)

    base = jnp.minimum(sid * STRIPE, N_C - STRIPE)

    @pl.loop(0, N_ZERO)
    def _(g):
        off = jnp.minimum(g * GSZ, STRIPE - GSZ)
        pltpu.sync_copy(zbuf, acc.at[pl.ds(base + off, GSZ)])

    plsc.subcore_barrier()

    gb0 = sid * GROUPS_PER_TILE
    pltpu.async_copy(epk.at[pl.ds(gb0, CHUNK_G)], eb0, sem_e)
    pltpu.async_copy(vpk.at[pl.ds(gb0, CHUNK_G)], vb0, sem_e)

    def do_chunk(c, ebuf, ebnext, vbuf, vbnext):
        def scat_wait(b):
            pltpu.make_async_copy(
                rbs[b], acc.at[ebuf.at[0, 0]], ssem[b]).wait()

        pltpu.make_async_copy(epk.at[pl.ds(0, CHUNK_G)], ebuf, sem_e).wait()
        pltpu.make_async_copy(vpk.at[pl.ds(0, CHUNK_G)], vbuf, sem_e).wait()

        @pl.when(c + 1 < CHUNKS)
        def _():
            nb = gb0 + (c + 1) * CHUNK_G
            pltpu.async_copy(epk.at[pl.ds(nb, CHUNK_G)], ebnext, sem_e)
            pltpu.async_copy(vpk.at[pl.ds(nb, CHUNK_G)], vbnext, sem_e)

        pltpu.async_copy(ego.at[ebuf.at[0, 1]], rbs[0], gsem[0])
        pltpu.async_copy(ego.at[ebuf.at[1, 1]], rbs[1], gsem[1])

        @pl.loop(0, CHUNK_G // 4)
        def _(i):
            for k in range(4):
                q = 4 * i + k
                nxt = (k + 2) % 4
                prv = (k + 3) % 4
                if k == 0:
                    @pl.when(i > 0)
                    def _():
                        scat_wait(prv)
                else:
                    scat_wait(prv)

                @pl.when(q + 2 < CHUNK_G)
                def _():
                    pltpu.async_copy(
                        ego.at[ebuf.at[q + 2, 1]], rbs[nxt], gsem[nxt])

                pltpu.make_async_copy(
                    ego.at[pl.ds(0, GSZ)], rbs[k], gsem[k]).wait()
                _scale(vbuf, q, rbs[k])
                pltpu.async_copy(
                    rbs[k], acc.at[ebuf.at[q, 0]], ssem[k], add=True)

        scat_wait(3)

    @pl.loop(0, CHUNKS // 2)
    def _(c2):
        do_chunk(2 * c2, eb0, eb1, vb0, vb1)
        do_chunk(2 * c2 + 1, eb1, eb0, vb1, vb0)

    plsc.subcore_barrier()
    pltpu.sync_copy(acc.at[pl.ds(base, STRIPE)], out.at[pl.ds(base, STRIPE)])


@functools.partial(
    pl.kernel,
    out_type=[jax.ShapeDtypeStruct((N_C, HD), jnp.float32)] * (2 * LAYERS_C),
    mesh=_mesh,
    scratch_types=[
        pltpu.VMEM_SHARED((N_C, HD), jnp.float32),
        pltpu.VMEM((CHUNK_G, 2, GSZ), jnp.int32),
        pltpu.VMEM((CHUNK_G, 2, GSZ), jnp.int32),
        pltpu.VMEM((CHUNK_G, GSZ), jnp.float32),
        pltpu.VMEM((CHUNK_G, GSZ), jnp.float32),
        pltpu.VMEM((GSZ, HD), jnp.float32),
        pltpu.VMEM((GSZ, HD), jnp.float32),
        pltpu.VMEM((GSZ, HD), jnp.float32),
        pltpu.VMEM((GSZ, HD), jnp.float32),
        pltpu.VMEM((GSZ, HD), jnp.float32),
        pltpu.SemaphoreType.DMA,
        pltpu.SemaphoreType.DMA,
        pltpu.SemaphoreType.DMA,
        pltpu.SemaphoreType.DMA,
        pltpu.SemaphoreType.DMA,
        pltpu.SemaphoreType.DMA,
        pltpu.SemaphoreType.DMA,
        pltpu.SemaphoreType.DMA,
        pltpu.SemaphoreType.DMA,
    ],
    compiler_params=_cparams,
)
def _spmm3(egoA, egoB, epk, vpk, oA1, oB1, oA2, oB2, oA3, oB3,
           acc, eb0, eb1, vb0, vb1, rb0, rb1, rb2, rb3, zbuf, sem_e,
           sg0, sg1, sg2, sg3, ss0, ss1, ss2, ss3):
    cid = lax.axis_index("c")
    sid = lax.axis_index("s")
    rbs = (rb0, rb1, rb2, rb3)
    gsem = (sg0, sg1, sg2, sg3)
    ssem = (ss0, ss1, ss2, ss3)

    @pl.when(cid == 0)
    def _():
        for src_t, dst_t in ((egoA, oA1), (oA1, oA2), (oA2, oA3)):
            _layer(sid, src_t, dst_t, epk, vpk, acc, eb0, eb1, vb0, vb1,
                   rbs, zbuf, sem_e, gsem, ssem)
            plsc.subcore_barrier()

    @pl.when(cid == 1)
    def _():
        for src_t, dst_t in ((egoB, oB1), (oB1, oB2), (oB2, oB3)):
            _layer(sid, src_t, dst_t, epk, vpk, acc, eb0, eb1, vb0, vb1,
                   rbs, zbuf, sem_e, gsem, ssem)
            plsc.subcore_barrier()


IDX_TOTAL = 3 * B_C
IDX_G = IDX_TOTAL // GSZ
IDX_G_PER_TILE = 8
IDX_TILES = IDX_G // IDX_G_PER_TILE


@functools.partial(
    pl.kernel,
    out_type=[jax.ShapeDtypeStruct((IDX_TOTAL, HD), jnp.float32)] * (2 * (LAYERS_C + 1)),
    mesh=_mesh,
    scratch_types=[
        pltpu.VMEM((IDX_G_PER_TILE, GSZ), jnp.int32),
        pltpu.VMEM((GSZ, HD), jnp.float32),
    ],
    compiler_params=_cparams,
)
def _gather8(t0, t1, t2, t3, t4, t5, t6, t7, idx_hbm,
             o0, o1, o2, o3, o4, o5, o6, o7, idxv, buf):
    cid = lax.axis_index("c")
    sid = lax.axis_index("s")
    w = cid * NS + sid

    @pl.when(w < IDX_TILES)
    def _():
        pltpu.sync_copy(
            idx_hbm.at[pl.ds(w * IDX_G_PER_TILE, IDX_G_PER_TILE)], idxv)
        for tab, out in ((t0, o0), (t1, o1), (t2, o2), (t3, o3),
                         (t4, o4), (t5, o5), (t6, o6), (t7, o7)):
            @pl.loop(0, IDX_G_PER_TILE)
            def _(j):
                pltpu.sync_copy(tab.at[idxv.at[j]], buf)
                pltpu.sync_copy(
                    buf, out.at[pl.ds((w * IDX_G_PER_TILE + j) * GSZ, GSZ)])


def kernel(users, pos_items, neg_items, edge_index, adj_vals, user_emb, item_emb):
    ego0 = jnp.concatenate([user_emb, item_emb], axis=0)
    row = edge_index[0].astype(jnp.int32)
    col = edge_index[1].astype(jnp.int32)
    pad = E_PAD - E_C
    rowp = jnp.pad(row, (0, pad)).reshape(G_TOTAL, GSZ)
    colp = jnp.pad(col, (0, pad)).reshape(G_TOTAL, GSZ)
    vpk = jnp.pad(adj_vals, (0, pad)).reshape(G_TOTAL, GSZ)
    epk = jnp.stack([rowp, colp], axis=1)

    egoA, egoB = ego0[:, :HD], ego0[:, HD:]
    oA1, oB1, oA2, oB2, oA3, oB3 = _spmm3(egoA, egoB, epk, vpk)
    halves = [(egoA, egoB), (oA1, oB1), (oA2, oB2), (oA3, oB3)]

    idx_all = jnp.concatenate([
        users.astype(jnp.int32),
        pos_items.astype(jnp.int32) + N_USER_C,
        neg_items.astype(jnp.int32) + N_USER_C,
    ]).reshape(IDX_G, GSZ)

    tabs = [h for pair in halves for h in pair]
    g = _gather8(*tabs, idx_all)
    cat = jnp.concatenate(
        [jnp.concatenate([g[2 * k], g[2 * k + 1]], axis=1)
         for k in range(LAYERS_C + 1)], axis=1)
    return (cat[:B_C], cat[B_C:2 * B_C], cat[2 * B_C:])
